# baseline (device time: 24243 ns/iter reference)
import jax
import jax.numpy as jnp
from jax import lax
from jax.experimental import pallas as pl
from jax.experimental.pallas import tpu as pltpu

N_CHUNKS = 32


def kernel(x):
    m, n = x.shape
    half_n = n // 2
    half_m = m // 2
    ch = half_m // N_CHUNKS

    def body(x_ref, out_ref, stage_ref, sx_sems, rx_sems, sy_sems, ry_sems):
        my_x = lax.axis_index("x")
        my_y = lax.axis_index("y")
        my_z = lax.axis_index("z")
        px = 1 - my_x
        s = my_y % 2
        py = my_y ^ 1

        barrier_sem = pltpu.get_barrier_semaphore()
        for dev in [(px, my_y, my_z), (my_x, py, my_z)]:
            pl.semaphore_signal(
                barrier_sem, inc=1,
                device_id=dev, device_id_type=pl.DeviceIdType.MESH,
            )

        x_rdmas = []
        for c in range(N_CHUNKS):
            stage_ref[pl.ds(c * ch, ch), :] = x_ref[
                pl.ds(s * half_m + c * ch, ch), pl.ds(px * half_n, half_n)
            ].astype(jnp.bfloat16)
            if c == 0:
                pl.semaphore_wait(barrier_sem, 2)
            rdma = pltpu.make_async_remote_copy(
                src_ref=stage_ref.at[pl.ds(c * ch, ch), :],
                dst_ref=out_ref.at[
                    pl.ds(my_x * m + s * half_m + c * ch, ch), :
                ],
                send_sem=sx_sems.at[c],
                recv_sem=rx_sems.at[c],
                device_id=(px, my_y, my_z),
                device_id_type=pl.DeviceIdType.MESH,
            )
            rdma.start()
            x_rdmas.append(rdma)

        out_ref[pl.ds(my_x * m, m), :] = x_ref[
            :, pl.ds(my_x * half_n, half_n)
        ].astype(jnp.bfloat16)

        y_rdmas = []
        for c in range(N_CHUNKS):
            x_rdmas[c].wait_recv()
            rows = pl.ds(px * m + s * half_m + c * ch, ch)
            fwd = pltpu.make_async_remote_copy(
                src_ref=out_ref.at[rows, :],
                dst_ref=out_ref.at[rows, :],
                send_sem=sy_sems.at[c],
                recv_sem=ry_sems.at[c],
                device_id=(my_x, py, my_z),
                device_id_type=pl.DeviceIdType.MESH,
            )
            fwd.start()
            y_rdmas.append(fwd)

        for c in range(N_CHUNKS):
            y_rdmas[c].wait_recv()
        for c in range(N_CHUNKS):
            x_rdmas[c].wait_send()
            y_rdmas[c].wait_send()

    return pl.pallas_call(
        body,
        out_shape=jax.ShapeDtypeStruct((2 * m, half_n), jnp.bfloat16),
        in_specs=[pl.BlockSpec(memory_space=pltpu.VMEM)],
        out_specs=pl.BlockSpec(memory_space=pltpu.VMEM),
        scratch_shapes=[
            pltpu.VMEM((half_m, half_n), jnp.bfloat16),
            pltpu.SemaphoreType.DMA((N_CHUNKS,)),
            pltpu.SemaphoreType.DMA((N_CHUNKS,)),
            pltpu.SemaphoreType.DMA((N_CHUNKS,)),
            pltpu.SemaphoreType.DMA((N_CHUNKS,)),
        ],
        compiler_params=pltpu.CompilerParams(collective_id=0),
    )(x)


# device time: 23955 ns/iter; 1.0120x vs baseline; 1.0120x over previous
import jax
import jax.numpy as jnp
from jax import lax
from jax.experimental import pallas as pl
from jax.experimental.pallas import tpu as pltpu

N_CHUNKS = 16


def kernel(x):
    m, n = x.shape
    half_n = n // 2
    half_m = m // 2
    ch = half_m // N_CHUNKS

    def body(x_ref, out_ref, stage_ref, sx_sems, rx_sems, sy_sems, ry_sems):
        my_x = lax.axis_index("x")
        my_y = lax.axis_index("y")
        my_z = lax.axis_index("z")
        px = 1 - my_x
        s = my_y % 2
        py = my_y ^ 1

        barrier_sem = pltpu.get_barrier_semaphore()
        for dev in [(px, my_y, my_z), (my_x, py, my_z)]:
            pl.semaphore_signal(
                barrier_sem, inc=1,
                device_id=dev, device_id_type=pl.DeviceIdType.MESH,
            )

        x_rdmas = []
        for c in range(N_CHUNKS):
            stage_ref[pl.ds(c * ch, ch), :] = x_ref[
                pl.ds(s * half_m + c * ch, ch), pl.ds(px * half_n, half_n)
            ].astype(jnp.bfloat16)
            if c == 0:
                pl.semaphore_wait(barrier_sem, 2)
            rdma = pltpu.make_async_remote_copy(
                src_ref=stage_ref.at[pl.ds(c * ch, ch), :],
                dst_ref=out_ref.at[
                    pl.ds(my_x * m + s * half_m + c * ch, ch), :
                ],
                send_sem=sx_sems.at[c],
                recv_sem=rx_sems.at[c],
                device_id=(px, my_y, my_z),
                device_id_type=pl.DeviceIdType.MESH,
            )
            rdma.start()
            x_rdmas.append(rdma)

        out_ref[pl.ds(my_x * m, m), :] = x_ref[
            :, pl.ds(my_x * half_n, half_n)
        ].astype(jnp.bfloat16)

        y_rdmas = []
        for c in range(N_CHUNKS):
            x_rdmas[c].wait_recv()
            rows = pl.ds(px * m + s * half_m + c * ch, ch)
            fwd = pltpu.make_async_remote_copy(
                src_ref=out_ref.at[rows, :],
                dst_ref=out_ref.at[rows, :],
                send_sem=sy_sems.at[c],
                recv_sem=ry_sems.at[c],
                device_id=(my_x, py, my_z),
                device_id_type=pl.DeviceIdType.MESH,
            )
            fwd.start()
            y_rdmas.append(fwd)

        for c in range(N_CHUNKS):
            y_rdmas[c].wait_recv()
        for c in range(N_CHUNKS):
            x_rdmas[c].wait_send()
            y_rdmas[c].wait_send()

    return pl.pallas_call(
        body,
        out_shape=jax.ShapeDtypeStruct((2 * m, half_n), jnp.bfloat16),
        in_specs=[pl.BlockSpec(memory_space=pltpu.VMEM)],
        out_specs=pl.BlockSpec(memory_space=pltpu.VMEM),
        scratch_shapes=[
            pltpu.VMEM((half_m, half_n), jnp.bfloat16),
            pltpu.SemaphoreType.DMA((N_CHUNKS,)),
            pltpu.SemaphoreType.DMA((N_CHUNKS,)),
            pltpu.SemaphoreType.DMA((N_CHUNKS,)),
            pltpu.SemaphoreType.DMA((N_CHUNKS,)),
        ],
        compiler_params=pltpu.CompilerParams(collective_id=0),
    )(x)
